# baseline (speedup 1.0000x reference)
import jax
import jax.numpy as jnp
from jax.experimental import pallas as pl
from jax.experimental.pallas import tpu as pltpu

_N = 4096
_TQ = 256
_W3 = 0.33
_LOG2E = 1.4426950408889634
_C0 = -_LOG2E / (2.0 * 0.02 * 0.02)
_C1 = -_LOG2E / (2.0 * 0.08 * 0.08)
_C2 = -_LOG2E / (2.0 * 0.32 * 0.32)


def _net_body(uq, vs, fea,
              w00, b00, w01, b01, w02, b02,
              w10, b10, w11, b11, w12, b12,
              w20, b20, w21, b21, w22, b22,
              wr, br, out,
              fA, fB, fSB):
    def cc(src, w, b, dst_ref, relu=True):
        cout = w.shape[1]
        y = jnp.dot(src.astype(jnp.bfloat16), w[...],
                    preferred_element_type=jnp.float32) + b[...]
        if relu:
            y = jnp.maximum(y, 0.0)
        dst_ref[:, :cout] = y

    def spatial(coef, cin, src_ref, dst_ref):
        fSB[:, :cin] = src_ref[:, :cin].astype(jnp.bfloat16)
        fSB[:, cin:cin + 1] = jnp.ones((_N, 1), jnp.bfloat16)

        def tile(i, c):
            r0 = i * _TQ
            qd = jnp.dot(uq[pl.ds(r0, _TQ), :], vs[...],
                         preferred_element_type=jnp.float32)
            t = jnp.minimum(qd * coef, 0.0)
            e = jnp.exp2(t)
            e16 = jnp.exp2(16.0 * t)
            e2 = e * e
            e4 = e2 * e2
            k = (e + e4) + e16
            num = jnp.dot(k.astype(jnp.bfloat16), fSB[:, :cin + 1],
                          preferred_element_type=jnp.float32)
            rs = num[:, cin:cin + 1]
            dst_ref[pl.ds(r0, _TQ), :cin] = (
                (_W3 * num[:, :cin]) / (_W3 * rs + 1e-8))
            return c

        jax.lax.fori_loop(0, _N // _TQ, tile, 0)

    cc(fea[...], w00, b00, fA)
    cc(fA[:, :8], w01, b01, fB)
    spatial(_C0, 16, fB, fA)
    cc(fA[:, :16], w02, b02, fB)
    spatial(_C1, 16, fB, fA)
    cc(fA[:, :16], w10, b10, fB)
    cc(fB[:, :32], w11, b11, fA)
    spatial(_C1, 32, fA, fB)
    cc(fB[:, :32], w12, b12, fA)
    spatial(_C2, 32, fA, fB)
    cc(fB[:, :32], w20, b20, fA)
    cc(fA[:, :64], w21, b21, fB)
    spatial(_C2, 64, fB, fA)
    cc(fA[:, :64], w22, b22, fB)
    y = jnp.dot(fB[:, :64].astype(jnp.bfloat16), wr[...],
                preferred_element_type=jnp.float32) + br[...]
    out[...] = y


def kernel(pc1, feature1, W00, b00, W01, b01, W02, b02, W10, b10, W11, b11,
           W12, b12, W20, b20, W21, b21, W22, b22, Wr, br):
    pc = pc1[0]
    fea = feature1[0]
    nrm2 = jnp.sum(pc * pc, axis=1, keepdims=True)
    hi = nrm2.astype(jnp.bfloat16).astype(jnp.float32)
    lo = nrm2 - hi
    ones = jnp.ones((_N, 1), jnp.float32)
    zero = jnp.zeros((_N, 1), jnp.float32)
    uq = jnp.concatenate([pc, ones, ones, hi, lo, zero],
                         axis=1).astype(jnp.bfloat16)
    vs = jnp.concatenate([-2.0 * pc, hi, lo, ones, ones, zero],
                         axis=1).astype(jnp.bfloat16).T
    wts = []
    for w, b in ((W00, b00), (W01, b01), (W02, b02), (W10, b10), (W11, b11),
                 (W12, b12), (W20, b20), (W21, b21), (W22, b22), (Wr, br)):
        wts.append(w.T.astype(jnp.bfloat16))
        wts.append(b[None, :])
    out = pl.pallas_call(
        _net_body,
        out_shape=jax.ShapeDtypeStruct((_N, 32), jnp.float32),
        scratch_shapes=[
            pltpu.VMEM((_N, 64), jnp.float32),
            pltpu.VMEM((_N, 64), jnp.float32),
            pltpu.VMEM((_N, 72), jnp.bfloat16),
        ],
    )(uq, vs, fea, *wts)
    return out[None]


# speedup vs baseline: 1.3607x; 1.3607x over previous
import numpy as np
import jax
import jax.numpy as jnp
from jax.experimental import pallas as pl
from jax.experimental.pallas import tpu as pltpu

_N = 4096
_TB = 512
_NT = _N // _TB
_W3 = 0.33
_LOG2E = 1.4426950408889634
_C0 = -_LOG2E / (2.0 * 0.02 * 0.02)
_C1 = -_LOG2E / (2.0 * 0.08 * 0.08)
_C2 = -_LOG2E / (2.0 * 0.32 * 0.32)

_PAIRS = [(i, j) for i in range(_NT) for j in range(i + 1, _NT)]
_NP = len(_PAIRS)
_NB = _NT + _NP


def _net_body(ii, jj, uq, vs, fea,
              w00, b00, w01, b01, w02, b02,
              w10, b10, w11, b11, w12, b12,
              w20, b20, w21, b21, w22, b22,
              wr, br, out,
              fA, fB, fSB, acc, kst):
    def cc(src, w, b, dst_ref, relu=True):
        cout = w.shape[1]
        y = jnp.dot(src.astype(jnp.bfloat16), w[...],
                    preferred_element_type=jnp.float32) + b[...]
        if relu:
            y = jnp.maximum(y, 0.0)
        dst_ref[:, :cout] = y

    def load_f(cin, src_ref):
        fSB[:, :cin] = src_ref[:, :cin].astype(jnp.bfloat16)
        fSB[:, cin:cin + 1] = jnp.ones((_N, 1), jnp.bfloat16)
        acc[:, :cin + 1] = jnp.zeros((_N, cin + 1), jnp.float32)

    def normalize(cin, dst_ref):
        num = acc[:, :cin]
        rs = acc[:, cin:cin + 1]
        dst_ref[:, :cin] = (_W3 * num) / (_W3 * rs + 1e-8)

    def apply_blk(kb, ri, rj, cin, both):
        acc[pl.ds(ri, _TB), :cin + 1] += jnp.dot(
            kb, fSB[pl.ds(rj, _TB), :cin + 1],
            preferred_element_type=jnp.float32)
        if both:
            acc[pl.ds(rj, _TB), :cin + 1] += jax.lax.dot_general(
                kb, fSB[pl.ds(ri, _TB), :cin + 1],
                (((0,), (0,)), ((), ())),
                preferred_element_type=jnp.float32)

    def spatial(coef, cin, src_ref, dst_ref, store):
        load_f(cin, src_ref)

        def kblock(ri, rj):
            qd = jnp.dot(uq[pl.ds(ri, _TB), :], vs[:, pl.ds(rj, _TB)],
                         preferred_element_type=jnp.float32)
            t = jnp.minimum(qd * coef, 0.0)
            e = jnp.exp2(t)
            e2 = e * e
            e4 = e2 * e2
            e8 = e4 * e4
            e16 = e8 * e8
            k = (e + e4) + e16
            return k.astype(jnp.bfloat16)

        def diag(i, c):
            r0 = i * _TB
            kb = kblock(r0, r0)
            if store:
                kst[pl.ds(i * _TB, _TB), :] = kb
            apply_blk(kb, r0, r0, cin, False)
            return c

        def offd(p, c):
            ri = ii[p] * _TB
            rj = jj[p] * _TB
            kb = kblock(ri, rj)
            if store:
                kst[pl.ds((_NT + p) * _TB, _TB), :] = kb
            apply_blk(kb, ri, rj, cin, True)
            return c

        jax.lax.fori_loop(0, _NT, diag, 0)
        jax.lax.fori_loop(0, _NP, offd, 0)
        normalize(cin, dst_ref)

    def spatial_reuse(cin, src_ref, dst_ref):
        load_f(cin, src_ref)

        def diag(i, c):
            r0 = i * _TB
            kb = kst[pl.ds(i * _TB, _TB), :]
            apply_blk(kb, r0, r0, cin, False)
            return c

        def offd(p, c):
            ri = ii[p] * _TB
            rj = jj[p] * _TB
            kb = kst[pl.ds((_NT + p) * _TB, _TB), :]
            apply_blk(kb, ri, rj, cin, True)
            return c

        jax.lax.fori_loop(0, _NT, diag, 0)
        jax.lax.fori_loop(0, _NP, offd, 0)
        normalize(cin, dst_ref)

    cc(fea[...], w00, b00, fA)
    cc(fA[:, :8], w01, b01, fB)
    spatial(_C0, 16, fB, fA, False)
    cc(fA[:, :16], w02, b02, fB)
    spatial(_C1, 16, fB, fA, True)
    cc(fA[:, :16], w10, b10, fB)
    cc(fB[:, :32], w11, b11, fA)
    spatial_reuse(32, fA, fB)
    cc(fB[:, :32], w12, b12, fA)
    spatial(_C2, 32, fA, fB, True)
    cc(fB[:, :32], w20, b20, fA)
    cc(fA[:, :64], w21, b21, fB)
    spatial_reuse(64, fB, fA)
    cc(fA[:, :64], w22, b22, fB)
    y = jnp.dot(fB[:, :64].astype(jnp.bfloat16), wr[...],
                preferred_element_type=jnp.float32) + br[...]
    out[...] = y


def kernel(pc1, feature1, W00, b00, W01, b01, W02, b02, W10, b10, W11, b11,
           W12, b12, W20, b20, W21, b21, W22, b22, Wr, br):
    pc = pc1[0]
    fea = feature1[0]
    nrm2 = jnp.sum(pc * pc, axis=1, keepdims=True)
    hi = nrm2.astype(jnp.bfloat16).astype(jnp.float32)
    lo = nrm2 - hi
    ones = jnp.ones((_N, 1), jnp.float32)
    zero = jnp.zeros((_N, 1), jnp.float32)
    uq = jnp.concatenate([pc, ones, ones, hi, lo, zero],
                         axis=1).astype(jnp.bfloat16)
    vs = jnp.concatenate([-2.0 * pc, hi, lo, ones, ones, zero],
                         axis=1).astype(jnp.bfloat16).T
    ii = jnp.asarray(np.array([p[0] for p in _PAIRS], np.int32))
    jj = jnp.asarray(np.array([p[1] for p in _PAIRS], np.int32))
    wts = []
    for w, b in ((W00, b00), (W01, b01), (W02, b02), (W10, b10), (W11, b11),
                 (W12, b12), (W20, b20), (W21, b21), (W22, b22), (Wr, br)):
        wts.append(w.T.astype(jnp.bfloat16))
        wts.append(b[None, :])
    out = pl.pallas_call(
        _net_body,
        out_shape=jax.ShapeDtypeStruct((_N, 32), jnp.float32),
        in_specs=[pl.BlockSpec(memory_space=pltpu.SMEM),
                  pl.BlockSpec(memory_space=pltpu.SMEM)] +
                 [pl.BlockSpec(memory_space=pltpu.VMEM)] * 23,
        out_specs=pl.BlockSpec(memory_space=pltpu.VMEM),
        scratch_shapes=[
            pltpu.VMEM((_N, 64), jnp.float32),
            pltpu.VMEM((_N, 64), jnp.float32),
            pltpu.VMEM((_N, 72), jnp.bfloat16),
            pltpu.VMEM((_N, 72), jnp.float32),
            pltpu.VMEM((_NB * _TB, _TB), jnp.bfloat16),
        ],
    )(ii, jj, uq, vs, fea, *wts)
    return out[None]
